# initial kernel scaffold (unmeasured)
import jax
import jax.numpy as jnp
from jax import lax
from jax.experimental import pallas as pl
from jax.experimental.pallas import tpu as pltpu


def kernel(
    x,
):
    def body(*refs):
        pass

    out_shape = jax.ShapeDtypeStruct(..., jnp.float32)
    return pl.pallas_call(body, out_shape=out_shape)(...)


# baseline (device time: 19326 ns/iter reference)
import jax
import jax.numpy as jnp
from jax import lax
from jax.experimental import pallas as pl
from jax.experimental.pallas import tpu as pltpu

N_Y = 4


def kernel(x):
    m_per, n = x.shape

    def body(x_ref, out_ref, comm_ref, send_sems, recv_sems):
        my_x = lax.axis_index("x")
        my_y = lax.axis_index("y")
        my_z = lax.axis_index("z")
        left = (my_y - 1) % N_Y
        right = (my_y + 1) % N_Y

        barrier_sem = pltpu.get_barrier_semaphore()
        for nbr in (left, right):
            pl.semaphore_signal(
                barrier_sem,
                inc=1,
                device_id=(my_x, nbr, my_z),
                device_id_type=pl.DeviceIdType.MESH,
            )
        pl.semaphore_wait(barrier_sem, 2)

        out_ref[pl.ds(my_y * m_per, m_per), :] = x_ref[:, :]
        comm_ref[0, :, :] = x_ref[:, :]

        for h in range(N_Y - 1):
            send_slot = h % 2
            recv_slot = (h + 1) % 2
            rdma = pltpu.make_async_remote_copy(
                src_ref=comm_ref.at[send_slot],
                dst_ref=comm_ref.at[recv_slot],
                send_sem=send_sems.at[send_slot],
                recv_sem=recv_sems.at[recv_slot],
                device_id=(my_x, right, my_z),
                device_id_type=pl.DeviceIdType.MESH,
            )
            rdma.start()
            rdma.wait()

            origin = (my_y - h - 1) % N_Y
            out_ref[pl.ds(origin * m_per, m_per), :] = comm_ref[recv_slot, :, :]

    return pl.pallas_call(
        body,
        out_shape=jax.ShapeDtypeStruct((N_Y * m_per, n), x.dtype),
        in_specs=[pl.BlockSpec(memory_space=pltpu.VMEM)],
        out_specs=pl.BlockSpec(memory_space=pltpu.VMEM),
        scratch_shapes=[
            pltpu.VMEM((2, m_per, n), x.dtype),
            pltpu.SemaphoreType.DMA((2,)),
            pltpu.SemaphoreType.DMA((2,)),
        ],
        compiler_params=pltpu.CompilerParams(collective_id=0),
    )(x)


# device time: 15971 ns/iter; 1.2101x vs baseline; 1.2101x over previous
import jax
import jax.numpy as jnp
from jax import lax
from jax.experimental import pallas as pl
from jax.experimental.pallas import tpu as pltpu

N_Y = 4


def kernel(x):
    m_per, n = x.shape
    half = m_per // 2

    def body(x_ref, out_ref, comm_ref, send_sems, recv_sems):
        my_x = lax.axis_index("x")
        my_y = lax.axis_index("y")
        my_z = lax.axis_index("z")
        left = (my_y - 1) % N_Y
        right = (my_y + 1) % N_Y
        far = (my_y + 2) % N_Y
        dev_left = (my_x, left, my_z)
        dev_right = (my_x, right, my_z)


        barrier_sem = pltpu.get_barrier_semaphore()
        for nbr in (dev_left, dev_right):
            pl.semaphore_signal(
                barrier_sem, inc=1,
                device_id=nbr, device_id_type=pl.DeviceIdType.MESH,
            )
        pl.semaphore_wait(barrier_sem, 2)

        send_r1 = pltpu.make_async_remote_copy(
            src_ref=x_ref,
            dst_ref=comm_ref.at[0],
            send_sem=send_sems.at[0],
            recv_sem=recv_sems.at[0],
            device_id=dev_right,
            device_id_type=pl.DeviceIdType.MESH,
        )
        send_l1 = pltpu.make_async_remote_copy(
            src_ref=x_ref,
            dst_ref=comm_ref.at[1],
            send_sem=send_sems.at[1],
            recv_sem=recv_sems.at[1],
            device_id=dev_left,
            device_id_type=pl.DeviceIdType.MESH,
        )
        send_r1.start()
        send_l1.start()

        out_ref[pl.ds(my_y * m_per, m_per), :] = x_ref[:, :]

        recv_l1 = pltpu.make_async_remote_copy(
            src_ref=x_ref,
            dst_ref=comm_ref.at[0],
            send_sem=send_sems.at[0],
            recv_sem=recv_sems.at[0],
            device_id=dev_right,
            device_id_type=pl.DeviceIdType.MESH,
        )
        recv_l1.wait_recv()
        send_r2 = pltpu.make_async_remote_copy(
            src_ref=comm_ref.at[0, pl.ds(0, half)],
            dst_ref=comm_ref.at[2, pl.ds(0, half)],
            send_sem=send_sems.at[2],
            recv_sem=recv_sems.at[2],
            device_id=dev_right,
            device_id_type=pl.DeviceIdType.MESH,
        )
        send_r2.start()
        out_ref[pl.ds(left * m_per, m_per), :] = comm_ref[0, :, :]

        recv_r1 = pltpu.make_async_remote_copy(
            src_ref=x_ref,
            dst_ref=comm_ref.at[1],
            send_sem=send_sems.at[1],
            recv_sem=recv_sems.at[1],
            device_id=dev_left,
            device_id_type=pl.DeviceIdType.MESH,
        )
        recv_r1.wait_recv()
        send_l2 = pltpu.make_async_remote_copy(
            src_ref=comm_ref.at[1, pl.ds(half, half)],
            dst_ref=comm_ref.at[2, pl.ds(half, half)],
            send_sem=send_sems.at[3],
            recv_sem=recv_sems.at[3],
            device_id=dev_left,
            device_id_type=pl.DeviceIdType.MESH,
        )
        send_l2.start()
        out_ref[pl.ds(right * m_per, m_per), :] = comm_ref[1, :, :]

        recv_l2 = pltpu.make_async_remote_copy(
            src_ref=x_ref.at[pl.ds(0, half)],
            dst_ref=comm_ref.at[2, pl.ds(0, half)],
            send_sem=send_sems.at[2],
            recv_sem=recv_sems.at[2],
            device_id=dev_right,
            device_id_type=pl.DeviceIdType.MESH,
        )
        recv_r2 = pltpu.make_async_remote_copy(
            src_ref=x_ref.at[pl.ds(0, half)],
            dst_ref=comm_ref.at[2, pl.ds(half, half)],
            send_sem=send_sems.at[3],
            recv_sem=recv_sems.at[3],
            device_id=dev_left,
            device_id_type=pl.DeviceIdType.MESH,
        )
        recv_l2.wait_recv()
        recv_r2.wait_recv()
        out_ref[pl.ds(far * m_per, m_per), :] = comm_ref[2, :, :]

        send_r1.wait_send()
        send_l1.wait_send()
        send_r2.wait_send()
        send_l2.wait_send()

    return pl.pallas_call(
        body,
        out_shape=jax.ShapeDtypeStruct((N_Y * m_per, n), x.dtype),
        in_specs=[pl.BlockSpec(memory_space=pltpu.VMEM)],
        out_specs=pl.BlockSpec(memory_space=pltpu.VMEM),
        scratch_shapes=[
            pltpu.VMEM((3, m_per, n), x.dtype),
            pltpu.SemaphoreType.DMA((4,)),
            pltpu.SemaphoreType.DMA((4,)),
        ],
        compiler_params=pltpu.CompilerParams(collective_id=0),
    )(x)
